# Initial kernel scaffold; baseline (speedup 1.0000x reference)
import functools
import math

import jax
import jax.numpy as jnp
from jax.experimental import pallas as pl
from jax.experimental.pallas import tpu as pltpu

_SMOOTHING = 0.1
_PAD = 0


def _body(tgt_ref, x_ref, out_ref, acc_ref, *, eps, cval):
    k = pl.program_id(0)
    br = x_ref.shape[0]
    tgt = tgt_ref[pl.ds(k * br, br), :]
    vf = (tgt != _PAD).astype(jnp.float32)
    xb = x_ref[...]
    xv = xb * vf
    s_all = jnp.sum(xv)
    s_col0 = jnp.sum(xv[:, 0])
    cols = jax.lax.broadcasted_iota(jnp.int32, xb.shape, 1)
    s_tgt = jnp.sum(jnp.where(cols == tgt, xv, 0.0))
    partial = (-eps) * s_all + eps * s_col0 + (eps - (1.0 - _SMOOTHING)) * s_tgt \
        + cval * jnp.sum(vf)

    @pl.when(k == 0)
    def _():
        acc_ref[0] = 0.0

    acc_ref[0] += partial

    @pl.when(k == pl.num_programs(0) - 1)
    def _():
        out_ref[0, 0] = acc_ref[0]


def kernel(x, target):
    n, v = x.shape
    eps = _SMOOTHING / (v - 2)
    cval = _SMOOTHING * math.log(eps) + (1.0 - _SMOOTHING) * math.log(1.0 - _SMOOTHING)
    br = 16 if n % 16 == 0 else 1
    grid = n // br
    tgt2d = target.astype(jnp.int32).reshape(n, 1)
    out = pl.pallas_call(
        functools.partial(_body, eps=eps, cval=cval),
        grid=(grid,),
        in_specs=[
            pl.BlockSpec((n, 1), lambda k: (0, 0)),
            pl.BlockSpec((br, v), lambda k: (k, 0)),
        ],
        out_specs=pl.BlockSpec((1, 1), lambda k: (0, 0)),
        out_shape=jax.ShapeDtypeStruct((1, 1), jnp.float32),
        scratch_shapes=[pltpu.SMEM((1,), jnp.float32)],
        compiler_params=pltpu.CompilerParams(
            dimension_semantics=("arbitrary",),
        ),
    )(tgt2d, x)
    return out.reshape(())


# speedup vs baseline: 1.6957x; 1.6957x over previous
import functools
import math

import jax
import jax.numpy as jnp
from jax.experimental import pallas as pl
from jax.experimental.pallas import tpu as pltpu

_SMOOTHING = 0.1
_PAD = 0


def _body(tgt_ref, x_ref, out_ref, acc_ref, *, eps, cval):
    k = pl.program_id(0)
    br = x_ref.shape[0]
    tgt = tgt_ref[pl.ds(k * br, br), :]
    vf = (tgt != _PAD).astype(jnp.float32)
    xb = x_ref[...]
    xv = xb * vf
    s_all = jnp.sum(xv)
    s_col0 = jnp.sum(xv[:, 0])
    cols = jax.lax.broadcasted_iota(jnp.int32, xb.shape, 1)
    s_tgt = jnp.sum(jnp.where(cols == tgt, xv, 0.0))
    partial = (-eps) * s_all + eps * s_col0 + (eps - (1.0 - _SMOOTHING)) * s_tgt \
        + cval * jnp.sum(vf)

    @pl.when(k == 0)
    def _():
        acc_ref[0] = 0.0

    acc_ref[0] += partial

    @pl.when(k == pl.num_programs(0) - 1)
    def _():
        out_ref[0, 0] = acc_ref[0]


def kernel(x, target):
    n, v = x.shape
    eps = _SMOOTHING / (v - 2)
    cval = _SMOOTHING * math.log(eps) + (1.0 - _SMOOTHING) * math.log(1.0 - _SMOOTHING)
    br = 16 if n % 16 == 0 else 1
    grid = n // br
    tgt2d = target.astype(jnp.int32).reshape(n, 1)
    out = pl.pallas_call(
        functools.partial(_body, eps=eps, cval=cval),
        grid=(grid,),
        in_specs=[
            pl.BlockSpec((n, 1), lambda k: (0, 0)),
            pl.BlockSpec((br, v), lambda k: (k, 0)),
        ],
        out_specs=pl.BlockSpec(memory_space=pltpu.SMEM),
        out_shape=jax.ShapeDtypeStruct((1, 1), jnp.float32),
        scratch_shapes=[pltpu.SMEM((1,), jnp.float32)],
        compiler_params=pltpu.CompilerParams(
            dimension_semantics=("arbitrary",),
        ),
    )(tgt2d, x)
    return out.reshape(())
